# Initial kernel scaffold; baseline (speedup 1.0000x reference)
import jax
import jax.numpy as jnp
from jax.experimental import pallas as pl


def kernel(etype, coeff, W):
    raise NotImplementedError("write your pallas kernel here")


# speedup vs baseline: 2.3103x; 2.3103x over previous
import functools

import jax
import jax.numpy as jnp
from jax import lax
from jax.experimental import pallas as pl
from jax.experimental.pallas import tpu as pltpu
from jax.experimental.pallas import tpu_sc as plsc


def _matmul_body(c_ref, w_ref, o_ref):
    o_ref[...] = jnp.dot(c_ref[...], w_ref[...],
                         preferred_element_type=jnp.float32)


def _basis_matmul(coeff, W):
    num_rels, _ = coeff.shape
    hidden = W.shape[1]
    return pl.pallas_call(
        _matmul_body,
        out_shape=jax.ShapeDtypeStruct((num_rels, hidden), jnp.float32),
    )(coeff, W)


@functools.cache
def _make_gather(E, D, NC, NS, chunk, nchunks):
    NW = NC * NS
    per = E // NW
    mesh = plsc.VectorSubcoreMesh(core_axis_name="c", subcore_axis_name="s")

    @functools.partial(
        pl.kernel,
        mesh=mesh,
        compiler_params=pltpu.CompilerParams(use_tc_tiling_on_sc=False),
        out_type=jax.ShapeDtypeStruct((E, D), jnp.float32),
        scratch_types=[
            pltpu.VMEM((nchunks, chunk), jnp.int32),
            pltpu.VMEM((chunk, D), jnp.float32),
            pltpu.VMEM((chunk, D), jnp.float32),
            pltpu.SemaphoreType.DMA,
            pltpu.SemaphoreType.DMA,
        ],
    )
    def gather_kernel(idx_hbm, table_hbm, out_hbm, idx_v, buf0, buf1,
                      sem0, sem1):
        wid = lax.axis_index("s") * NC + lax.axis_index("c")
        base = wid * per
        pltpu.sync_copy(idx_hbm.at[wid], idx_v)

        pltpu.async_copy(table_hbm.at[idx_v.at[0]], buf0, sem0)

        def body(j, carry):
            def step(cur_buf, cur_sem, nxt_buf, nxt_sem):
                @pl.when(j + 1 < nchunks)
                def _():
                    pltpu.async_copy(table_hbm.at[idx_v.at[j + 1]],
                                     nxt_buf, nxt_sem)
                pltpu.make_async_copy(table_hbm.at[idx_v.at[j]],
                                      cur_buf, cur_sem).wait()
                pltpu.sync_copy(cur_buf,
                                out_hbm.at[pl.ds(base + j * chunk, chunk)])

            @pl.when(j % 2 == 0)
            def _():
                step(buf0, sem0, buf1, sem1)

            @pl.when(j % 2 == 1)
            def _():
                step(buf1, sem1, buf0, sem0)

            return carry

        lax.fori_loop(0, nchunks, body, 0)

    return gather_kernel


def kernel(etype, coeff, W):
    E = etype.shape[0]
    D = W.shape[1]
    info = plsc.get_sparse_core_info()
    NC, NS = info.num_cores, info.num_subcores
    NW = NC * NS
    per = E // NW
    assert per * NW == E
    chunk = 8
    for c in range(8, 129, 8):
        if per % c == 0:
            chunk = c
    nchunks = per // chunk

    w = _basis_matmul(coeff, W)
    idx = etype.astype(jnp.int32).reshape(NW, nchunks, chunk)
    return _make_gather(E, D, NC, NS, chunk, nchunks)(idx, w)


# speedup vs baseline: 2.4261x; 1.0501x over previous
import functools

import jax
import jax.numpy as jnp
from jax import lax
from jax.experimental import pallas as pl
from jax.experimental.pallas import tpu as pltpu
from jax.experimental.pallas import tpu_sc as plsc


def _matmul_body(c_ref, w_ref, o_ref):
    o_ref[...] = jnp.dot(c_ref[...], w_ref[...],
                         preferred_element_type=jnp.float32)


def _basis_matmul(coeff, W):
    num_rels, _ = coeff.shape
    hidden = W.shape[1]
    return pl.pallas_call(
        _matmul_body,
        out_shape=jax.ShapeDtypeStruct((num_rels, hidden), jnp.float32),
    )(coeff, W)


@functools.cache
def _make_gather(E, D, NC, NS, chunk, nchunks):
    NW = NC * NS
    per = E // NW
    mesh = plsc.VectorSubcoreMesh(core_axis_name="c", subcore_axis_name="s")

    @functools.partial(
        pl.kernel,
        mesh=mesh,
        compiler_params=pltpu.CompilerParams(use_tc_tiling_on_sc=False),
        out_type=jax.ShapeDtypeStruct((E, D), jnp.float32),
        scratch_types=[
            pltpu.VMEM((nchunks, chunk), jnp.int32),
            pltpu.VMEM((chunk, D), jnp.float32),
            pltpu.VMEM((chunk, D), jnp.float32),
            pltpu.SemaphoreType.DMA,
            pltpu.SemaphoreType.DMA,
        ],
    )
    def gather_kernel(idx_hbm, table_hbm, out_hbm, idx_v, buf0, buf1,
                      sem0, sem1):
        wid = lax.axis_index("s") * NC + lax.axis_index("c")
        base = wid * per
        pltpu.sync_copy(idx_hbm.at[wid], idx_v)

        pltpu.async_copy(table_hbm.at[idx_v.at[0]], buf0, sem0)

        def body(j, carry):
            def step(cur_buf, cur_sem, nxt_buf, nxt_sem):
                @pl.when(j + 1 < nchunks)
                def _():
                    pltpu.async_copy(table_hbm.at[idx_v.at[j + 1]],
                                     nxt_buf, nxt_sem)
                pltpu.make_async_copy(table_hbm.at[idx_v.at[j]],
                                      cur_buf, cur_sem).wait()
                pltpu.sync_copy(cur_buf,
                                out_hbm.at[pl.ds(base + j * chunk, chunk)])

            @pl.when(j % 2 == 0)
            def _():
                step(buf0, sem0, buf1, sem1)

            @pl.when(j % 2 == 1)
            def _():
                step(buf1, sem1, buf0, sem0)

            return carry

        lax.fori_loop(0, nchunks, body, 0)

    return gather_kernel


def kernel(etype, coeff, W):
    E = etype.shape[0]
    D = W.shape[1]
    info = plsc.get_sparse_core_info()
    NC, NS = info.num_cores, info.num_subcores
    NW = NC * NS
    per = E // NW
    assert per * NW == E
    chunk = 8
    for c in range(8, 257, 8):
        if per % c == 0:
            chunk = c
    nchunks = per // chunk

    w = _basis_matmul(coeff, W)
    idx = etype.astype(jnp.int32).reshape(NW, nchunks, chunk)
    return _make_gather(E, D, NC, NS, chunk, nchunks)(idx, w)


# speedup vs baseline: 2.4268x; 1.0003x over previous
import functools

import jax
import jax.numpy as jnp
from jax import lax
from jax.experimental import pallas as pl
from jax.experimental.pallas import tpu as pltpu
from jax.experimental.pallas import tpu_sc as plsc


def _matmul_body(c_ref, w_ref, o_ref):
    o_ref[...] = jnp.dot(c_ref[...], w_ref[...],
                         preferred_element_type=jnp.float32)


def _basis_matmul(coeff, W):
    num_rels, _ = coeff.shape
    hidden = W.shape[1]
    return pl.pallas_call(
        _matmul_body,
        out_shape=jax.ShapeDtypeStruct((num_rels, hidden), jnp.float32),
    )(coeff, W)


@functools.cache
def _make_gather(E, D, NC, NS, chunk, nchunks):
    NW = NC * NS
    per = E // NW
    mesh = plsc.VectorSubcoreMesh(core_axis_name="c", subcore_axis_name="s")

    @functools.partial(
        pl.kernel,
        mesh=mesh,
        compiler_params=pltpu.CompilerParams(use_tc_tiling_on_sc=False),
        out_type=jax.ShapeDtypeStruct((E, D), jnp.float32),
        scratch_types=[
            pltpu.VMEM((per,), jnp.int32),
            pltpu.VMEM((chunk, D), jnp.float32),
            pltpu.VMEM((chunk, D), jnp.float32),
            pltpu.SemaphoreType.DMA,
            pltpu.SemaphoreType.DMA,
        ],
    )
    def gather_kernel(idx_hbm, table_hbm, out_hbm, idx_v, buf0, buf1,
                      sem0, sem1):
        wid = lax.axis_index("s") * NC + lax.axis_index("c")
        base = wid * per
        pltpu.sync_copy(idx_hbm.at[pl.ds(base, per)], idx_v)

        pltpu.async_copy(table_hbm.at[idx_v.at[pl.ds(0, chunk)]], buf0, sem0)

        def body(j, carry):
            def step(cur_buf, cur_sem, nxt_buf, nxt_sem):
                @pl.when(j + 1 < nchunks)
                def _():
                    pltpu.async_copy(
                        table_hbm.at[idx_v.at[pl.ds((j + 1) * chunk, chunk)]],
                        nxt_buf, nxt_sem)
                pltpu.make_async_copy(
                    table_hbm.at[idx_v.at[pl.ds(j * chunk, chunk)]],
                    cur_buf, cur_sem).wait()
                pltpu.sync_copy(cur_buf,
                                out_hbm.at[pl.ds(base + j * chunk, chunk)])

            @pl.when(j % 2 == 0)
            def _():
                step(buf0, sem0, buf1, sem1)

            @pl.when(j % 2 == 1)
            def _():
                step(buf1, sem1, buf0, sem0)

            return carry

        lax.fori_loop(0, nchunks, body, 0)

    return gather_kernel


def kernel(etype, coeff, W):
    E = etype.shape[0]
    D = W.shape[1]
    info = plsc.get_sparse_core_info()
    NC, NS = info.num_cores, info.num_subcores
    NW = NC * NS
    per = E // NW
    assert per * NW == E
    chunk = 8
    for c in range(8, 257, 8):
        if per % c == 0:
            chunk = c
    nchunks = per // chunk

    w = _basis_matmul(coeff, W)
    idx = etype.astype(jnp.int32)
    return _make_gather(E, D, NC, NS, chunk, nchunks)(idx, w)
